# Initial kernel scaffold; baseline (speedup 1.0000x reference)
import jax
import jax.numpy as jnp
from jax.experimental import pallas as pl


def kernel(x, edge_index, edge_attr, batch, W1, b1, W2, b2, Wa, ba, Wc, bc):
    raise NotImplementedError("write your pallas kernel here")


# speedup vs baseline: 12.1891x; 12.1891x over previous
import functools

import jax
import jax.numpy as jnp
from jax import lax
from jax.experimental import pallas as pl
from jax.experimental.pallas import tpu as pltpu
from jax.experimental.pallas import tpu_sc as plsc

NC = 2
NS = 16
NW = NC * NS
LANES = 16
CHUNK = 128
G_SEG = 16
DEG_W = 16


def _zero16():
    return jnp.zeros((LANES,), jnp.float32)


def _make_deg_kernel(N, T):
    n_chunks = T // CHUNK
    rows_per_sub = N // NS

    mesh = plsc.VectorSubcoreMesh(core_axis_name="c", subcore_axis_name="s",
                                  num_cores=NC, num_subcores=NS)

    @functools.partial(
        pl.kernel,
        out_type=jax.ShapeDtypeStruct((NC, NS, N // NS, DEG_W), jnp.float32),
        mesh=mesh,
        scratch_types=[
            pltpu.VMEM((CHUNK,), jnp.int32),
            pltpu.VMEM((CHUNK,), jnp.float32),
            pltpu.VMEM((CHUNK, DEG_W), jnp.float32),
            pltpu.VMEM((rows_per_sub, DEG_W), jnp.float32),
            pltpu.VMEM_SHARED((N, DEG_W), jnp.float32),
        ],
        compiler_params=pltpu.CompilerParams(use_tc_tiling_on_sc=False),
    )
    def deg_kernel(col_hbm, w_hbm, out_hbm, colv, wv, msgs, zbuf, acc):
        c = lax.axis_index("c")
        s = lax.axis_index("s")
        wid = s * NC + c
        z16 = _zero16()

        def zrow(i, carry):
            zbuf[i, :] = z16
            return carry
        lax.fori_loop(0, rows_per_sub, zrow, 0)
        pltpu.sync_copy(zbuf, acc.at[pl.ds(s * rows_per_sub, rows_per_sub)])
        plsc.subcore_barrier()

        base = wid * T

        def chunk(k, carry):
            off = base + k * CHUNK
            pltpu.sync_copy(col_hbm.at[pl.ds(off, CHUNK)], colv)
            pltpu.sync_copy(w_hbm.at[pl.ds(off, CHUNK)], wv)
            for g in range(CHUNK // LANES):
                w16 = jnp.abs(wv[pl.ds(g * LANES, LANES)])
                for e16 in range(LANES):
                    msgs[g * LANES + e16, :] = lax.broadcast_in_dim(
                        w16[e16], (LANES,), ())
            pltpu.sync_copy(msgs, acc.at[colv], add=True)
            return carry
        lax.fori_loop(0, n_chunks, chunk, 0)
        plsc.subcore_barrier()

        pltpu.sync_copy(
            acc.at[pl.ds(s * rows_per_sub, rows_per_sub)],
            out_hbm.at[c, s])

    return deg_kernel


def _make_layer_kernel(N, H, T):
    n_chunks = T // CHUNK
    rows_per_sub = N // NS
    zrows = rows_per_sub // 5
    jgroups = H // LANES

    mesh = plsc.VectorSubcoreMesh(core_axis_name="c", subcore_axis_name="s",
                                  num_cores=NC, num_subcores=NS)

    @functools.partial(
        pl.kernel,
        out_type=jax.ShapeDtypeStruct((NC, NS, N // NS, H), jnp.float32),
        mesh=mesh,
        scratch_types=[
            pltpu.VMEM((CHUNK,), jnp.int32),
            pltpu.VMEM((CHUNK,), jnp.int32),
            pltpu.VMEM((CHUNK,), jnp.float32),
            pltpu.VMEM((CHUNK, H), jnp.float32),
            pltpu.VMEM((zrows, H), jnp.float32),
            pltpu.VMEM_SHARED((N, H), jnp.float32),
            pltpu.SemaphoreType.DMA,
        ],
        compiler_params=pltpu.CompilerParams(use_tc_tiling_on_sc=False),
    )
    def layer_kernel(y_hbm, row_hbm, col_hbm, w_hbm, out_hbm,
                     rowv, colv, wv, rows, zbuf, acc, sem):
        c = lax.axis_index("c")
        s = lax.axis_index("s")
        wid = s * NC + c
        z16 = _zero16()

        def zrow(i, carry):
            for j in range(jgroups):
                zbuf[i, pl.ds(j * LANES, LANES)] = z16
            return carry
        lax.fori_loop(0, zrows, zrow, 0)
        for t in range(rows_per_sub // zrows):
            pltpu.sync_copy(
                zbuf, acc.at[pl.ds(s * rows_per_sub + t * zrows, zrows)])
        plsc.subcore_barrier()

        base = wid * T

        def chunk(k, carry):
            off = base + k * CHUNK
            pltpu.sync_copy(row_hbm.at[pl.ds(off, CHUNK)], rowv)
            pltpu.sync_copy(col_hbm.at[pl.ds(off, CHUNK)], colv)
            pltpu.sync_copy(w_hbm.at[pl.ds(off, CHUNK)], wv)
            pltpu.async_copy(y_hbm.at[rowv], rows, sem).wait()
            for g in range(CHUNK // LANES):
                w16 = jnp.abs(wv[pl.ds(g * LANES, LANES)])
                for e16 in range(LANES):
                    e = g * LANES + e16
                    bs = lax.broadcast_in_dim(w16[e16], (LANES,), ())
                    for j in range(jgroups):
                        rows[e, pl.ds(j * LANES, LANES)] = (
                            rows[e, pl.ds(j * LANES, LANES)] * bs)
            pltpu.sync_copy(rows, acc.at[colv], add=True)
            return carry
        lax.fori_loop(0, n_chunks, chunk, 0)
        plsc.subcore_barrier()

        pltpu.sync_copy(
            acc.at[pl.ds(s * rows_per_sub, rows_per_sub)],
            out_hbm.at[c, s])

    return layer_kernel


def _tc1_body(degp_ref, x_ref, w1_ref, dis_ref, y1_ref):
    dp = degp_ref[...]
    deg = dp[0][:, 0:1] + dp[1][:, 0:1] + 1.0
    dis = lax.rsqrt(deg)
    dis_ref[...] = dis
    y1_ref[...] = jnp.dot(x_ref[...], w1_ref[...],
                          preferred_element_type=jnp.float32,
                          precision=lax.Precision.HIGHEST) * dis


def _tc2_body(p_ref, y_ref, dis_ref, b_ref, w2_ref, y2_ref):
    dis = dis_ref[...]
    h = jnp.maximum(dis * (p_ref[0] + p_ref[1] + y_ref[...]) + b_ref[...], 0.0)
    y2_ref[...] = jnp.dot(h, w2_ref[...],
                          preferred_element_type=jnp.float32,
                          precision=lax.Precision.HIGHEST) * dis


def _tc3_body(p_ref, y_ref, dis_ref, b_ref, wa_ref, ba_ref, wc_ref, bc_ref,
              batch_ref, logits_ref, value_ref):
    dis = dis_ref[...]
    h = jnp.maximum(dis * (p_ref[0] + p_ref[1] + y_ref[...]) + b_ref[...], 0.0)
    logits_ref[...] = jnp.dot(h, wa_ref[...],
                              preferred_element_type=jnp.float32,
                              precision=lax.Precision.HIGHEST) + ba_ref[...]
    b = batch_ref[...]
    gi = lax.broadcasted_iota(jnp.int32, (G_SEG, 1), 0)
    onehot = (b == gi).astype(jnp.float32)
    sums = jnp.dot(onehot, h, preferred_element_type=jnp.float32,
                   precision=lax.Precision.HIGHEST)
    counts = jnp.sum(onehot, axis=1, keepdims=True)
    ge = sums / jnp.maximum(counts, 1.0)
    value_ref[...] = jnp.dot(ge, wc_ref[...],
                             preferred_element_type=jnp.float32,
                             precision=lax.Precision.HIGHEST) + bc_ref[...]


def kernel(x, edge_index, edge_attr, batch, W1, b1, W2, b2, Wa, ba, Wc, bc):
    N, F_IN = x.shape
    H = W1.shape[1]
    E = edge_index.shape[1]

    T = -(-E // (NW * CHUNK)) * CHUNK
    Ep = NW * T
    pad = Ep - E
    row = jnp.concatenate([edge_index[0], jnp.zeros((pad,), jnp.int32)])
    col = jnp.concatenate([edge_index[1], jnp.zeros((pad,), jnp.int32)])
    w = jnp.concatenate([edge_attr, jnp.zeros((pad,), jnp.float32)])

    deg_k = _make_deg_kernel(N, T)
    layer_k = _make_layer_kernel(N, H, T)

    degp = deg_k(col, w).reshape(NC, N, DEG_W)

    dis, y1 = pl.pallas_call(
        _tc1_body,
        out_shape=[jax.ShapeDtypeStruct((N, 1), jnp.float32),
                   jax.ShapeDtypeStruct((N, H), jnp.float32)],
    )(degp, x, W1)

    p1 = layer_k(y1, row, col, w).reshape(NC, N, H)

    y2 = pl.pallas_call(
        _tc2_body,
        out_shape=jax.ShapeDtypeStruct((N, H), jnp.float32),
    )(p1, y1, dis, b1.reshape(1, H), W2)

    p2 = layer_k(y2, row, col, w).reshape(NC, N, H)

    logits2, value = pl.pallas_call(
        _tc3_body,
        out_shape=[jax.ShapeDtypeStruct((N, 1), jnp.float32),
                   jax.ShapeDtypeStruct((G_SEG, 1), jnp.float32)],
    )(p2, y2, dis, b2.reshape(1, H), Wa, ba.reshape(1, 1), Wc,
      bc.reshape(1, 1), batch.reshape(1, N))

    return (logits2.reshape(N), value)


# speedup vs baseline: 16.0271x; 1.3149x over previous
import functools

import jax
import jax.numpy as jnp
from jax import lax
from jax.experimental import pallas as pl
from jax.experimental.pallas import tpu as pltpu
from jax.experimental.pallas import tpu_sc as plsc

NC = 2
NS = 16
NW = NC * NS
LANES = 16
CHUNK = 128
G_SEG = 16
DEG_W = 16


def _zero16():
    return jnp.zeros((LANES,), jnp.float32)


def _make_deg_kernel(N, T):
    nb = T // CHUNK
    npairs = nb // 2
    rows_per_sub = N // NS
    zrows = rows_per_sub // 5

    mesh = plsc.VectorSubcoreMesh(core_axis_name="c", subcore_axis_name="s",
                                  num_cores=NC, num_subcores=NS)

    @functools.partial(
        pl.kernel,
        out_type=jax.ShapeDtypeStruct((NC, NS, N // NS, DEG_W), jnp.float32),
        mesh=mesh,
        scratch_types=[
            pltpu.VMEM((nb, CHUNK), jnp.int32),
            pltpu.VMEM((nb, CHUNK), jnp.float32),
            pltpu.VMEM((CHUNK, DEG_W), jnp.float32),
            pltpu.VMEM((CHUNK, DEG_W), jnp.float32),
            pltpu.VMEM((zrows, DEG_W), jnp.float32),
            pltpu.VMEM_SHARED((N, DEG_W), jnp.float32),
            pltpu.SemaphoreType.DMA,
            pltpu.SemaphoreType.DMA,
        ],
        compiler_params=pltpu.CompilerParams(use_tc_tiling_on_sc=False),
    )
    def deg_kernel(col_hbm, w_hbm, out_hbm, col2, w2, msgs0, msgs1, zbuf,
                   acc, ss0, ss1):
        c = lax.axis_index("c")
        s = lax.axis_index("s")
        wid = s * NC + c
        z16 = _zero16()

        pltpu.sync_copy(col_hbm.at[wid], col2)
        pltpu.sync_copy(w_hbm.at[wid], w2)

        def zrow(i, carry):
            zbuf[i, :] = z16
            return carry
        lax.fori_loop(0, zrows, zrow, 0)
        for t in range(rows_per_sub // zrows):
            pltpu.sync_copy(
                zbuf, acc.at[pl.ds(s * rows_per_sub + t * zrows, zrows)])
        plsc.subcore_barrier()

        def build(kc, msgsb):
            for g in range(CHUNK // LANES):
                w16 = jnp.abs(w2[kc, pl.ds(g * LANES, LANES)])
                for e16 in range(LANES):
                    msgsb[g * LANES + e16, :] = lax.broadcast_in_dim(
                        w16[e16], (LANES,), ())

        def pair(k2, carry):
            a = 2 * k2

            @pl.when(k2 > 0)
            def _():
                pltpu.make_async_copy(msgs0, acc.at[col2.at[0]], ss0).wait()
            build(a, msgs0)
            pltpu.async_copy(msgs0, acc.at[col2.at[a]], ss0, add=True)

            @pl.when(k2 > 0)
            def _():
                pltpu.make_async_copy(msgs1, acc.at[col2.at[0]], ss1).wait()
            build(a + 1, msgs1)
            pltpu.async_copy(msgs1, acc.at[col2.at[a + 1]], ss1, add=True)
            return carry
        lax.fori_loop(0, npairs, pair, 0)
        pltpu.make_async_copy(msgs0, acc.at[col2.at[0]], ss0).wait()
        pltpu.make_async_copy(msgs1, acc.at[col2.at[0]], ss1).wait()
        plsc.subcore_barrier()

        pltpu.sync_copy(
            acc.at[pl.ds(s * rows_per_sub, rows_per_sub)],
            out_hbm.at[c, s])

    return deg_kernel


def _make_layer_kernel(N, H, T):
    nb = T // CHUNK
    npairs = nb // 2
    rows_per_sub = N // NS
    zrows = rows_per_sub // 5
    jgroups = H // LANES

    mesh = plsc.VectorSubcoreMesh(core_axis_name="c", subcore_axis_name="s",
                                  num_cores=NC, num_subcores=NS)

    @functools.partial(
        pl.kernel,
        out_type=jax.ShapeDtypeStruct((NC, NS, N // NS, H), jnp.float32),
        mesh=mesh,
        scratch_types=[
            pltpu.VMEM((nb, CHUNK), jnp.int32),
            pltpu.VMEM((nb, CHUNK), jnp.int32),
            pltpu.VMEM((nb, CHUNK), jnp.float32),
            pltpu.VMEM((CHUNK, H), jnp.float32),
            pltpu.VMEM((CHUNK, H), jnp.float32),
            pltpu.VMEM((CHUNK, H), jnp.float32),
            pltpu.VMEM((CHUNK, H), jnp.float32),
            pltpu.VMEM((zrows, H), jnp.float32),
            pltpu.VMEM_SHARED((N, H), jnp.float32),
            pltpu.SemaphoreType.DMA,
            pltpu.SemaphoreType.DMA,
            pltpu.SemaphoreType.DMA,
            pltpu.SemaphoreType.DMA,
        ],
        compiler_params=pltpu.CompilerParams(use_tc_tiling_on_sc=False),
    )
    def layer_kernel(y_hbm, row_hbm, col_hbm, w_hbm, out_hbm,
                     row2, col2, w2, rows0, rows1, msgs0, msgs1, zbuf, acc,
                     gs0, gs1, ss0, ss1):
        c = lax.axis_index("c")
        s = lax.axis_index("s")
        wid = s * NC + c
        z16 = _zero16()

        pltpu.sync_copy(row_hbm.at[wid], row2)
        pltpu.sync_copy(col_hbm.at[wid], col2)
        pltpu.sync_copy(w_hbm.at[wid], w2)

        def zrow(i, carry):
            for j in range(jgroups):
                zbuf[i, pl.ds(j * LANES, LANES)] = z16
            return carry
        lax.fori_loop(0, zrows, zrow, 0)
        for t in range(rows_per_sub // zrows):
            pltpu.sync_copy(
                zbuf, acc.at[pl.ds(s * rows_per_sub + t * zrows, zrows)])
        plsc.subcore_barrier()

        def scale(kc, rowsb, msgsb):
            for g in range(CHUNK // LANES):
                w16 = jnp.abs(w2[kc, pl.ds(g * LANES, LANES)])
                for e16 in range(LANES):
                    e = g * LANES + e16
                    bs = lax.broadcast_in_dim(w16[e16], (LANES,), ())
                    for j in range(jgroups):
                        msgsb[e, pl.ds(j * LANES, LANES)] = (
                            rowsb[e, pl.ds(j * LANES, LANES)] * bs)

        pltpu.async_copy(y_hbm.at[row2.at[0]], rows0, gs0)
        pltpu.async_copy(y_hbm.at[row2.at[1]], rows1, gs1)

        def half(k2, a, rowsb, msgsb, gsem, ssem):
            pltpu.make_async_copy(y_hbm.at[row2.at[0]], rowsb, gsem).wait()

            @pl.when(k2 > 0)
            def _():
                pltpu.make_async_copy(msgsb, acc.at[col2.at[0]], ssem).wait()
            scale(a, rowsb, msgsb)

            @pl.when(k2 < npairs - 1)
            def _():
                pltpu.async_copy(y_hbm.at[row2.at[a + 2]], rowsb, gsem)
            pltpu.async_copy(msgsb, acc.at[col2.at[a]], ssem, add=True)

        def pair(k2, carry):
            a = 2 * k2
            half(k2, a, rows0, msgs0, gs0, ss0)
            half(k2, a + 1, rows1, msgs1, gs1, ss1)
            return carry
        lax.fori_loop(0, npairs, pair, 0)
        pltpu.make_async_copy(msgs0, acc.at[col2.at[0]], ss0).wait()
        pltpu.make_async_copy(msgs1, acc.at[col2.at[0]], ss1).wait()
        plsc.subcore_barrier()

        pltpu.sync_copy(
            acc.at[pl.ds(s * rows_per_sub, rows_per_sub)],
            out_hbm.at[c, s])

    return layer_kernel


def _tc1_body(degp_ref, x_ref, w1_ref, dis_ref, y1_ref):
    dp = degp_ref[...]
    deg = dp[0][:, 0:1] + dp[1][:, 0:1] + 1.0
    dis = lax.rsqrt(deg)
    dis_ref[...] = dis
    y1_ref[...] = jnp.dot(x_ref[...], w1_ref[...],
                          preferred_element_type=jnp.float32,
                          precision=None) * dis


def _tc2_body(p_ref, y_ref, dis_ref, b_ref, w2_ref, y2_ref):
    dis = dis_ref[...]
    h = jnp.maximum(dis * (p_ref[0] + p_ref[1] + y_ref[...]) + b_ref[...], 0.0)
    y2_ref[...] = jnp.dot(h, w2_ref[...],
                          preferred_element_type=jnp.float32,
                          precision=None) * dis


def _tc3_body(p_ref, y_ref, dis_ref, b_ref, wa_ref, ba_ref, wc_ref, bc_ref,
              batch_ref, logits_ref, value_ref):
    dis = dis_ref[...]
    h = jnp.maximum(dis * (p_ref[0] + p_ref[1] + y_ref[...]) + b_ref[...], 0.0)
    logits_ref[...] = jnp.dot(h, wa_ref[...],
                              preferred_element_type=jnp.float32,
                              precision=None) + ba_ref[...]
    b = batch_ref[...]
    gi = lax.broadcasted_iota(jnp.int32, (G_SEG, 1), 0)
    onehot = (b == gi).astype(jnp.float32)
    sums = jnp.dot(onehot, h, preferred_element_type=jnp.float32,
                   precision=None)
    counts = jnp.sum(onehot, axis=1, keepdims=True)
    ge = sums / jnp.maximum(counts, 1.0)
    value_ref[...] = jnp.dot(ge, wc_ref[...],
                             preferred_element_type=jnp.float32,
                             precision=None) + bc_ref[...]


def kernel(x, edge_index, edge_attr, batch, W1, b1, W2, b2, Wa, ba, Wc, bc):
    N, F_IN = x.shape
    H = W1.shape[1]
    E = edge_index.shape[1]

    T = -(-E // (NW * 2 * CHUNK)) * 2 * CHUNK
    Ep = NW * T
    pad = Ep - E
    nb = T // CHUNK
    row = jnp.concatenate([edge_index[0], jnp.zeros((pad,), jnp.int32)])
    col = jnp.concatenate([edge_index[1], jnp.zeros((pad,), jnp.int32)])
    w = jnp.concatenate([edge_attr, jnp.zeros((pad,), jnp.float32)])
    row = row.reshape(NW, nb, CHUNK)
    col = col.reshape(NW, nb, CHUNK)
    w = w.reshape(NW, nb, CHUNK)

    deg_k = _make_deg_kernel(N, T)
    layer_k = _make_layer_kernel(N, H, T)

    degp = deg_k(col, w).reshape(NC, N, DEG_W)

    dis, y1 = pl.pallas_call(
        _tc1_body,
        out_shape=[jax.ShapeDtypeStruct((N, 1), jnp.float32),
                   jax.ShapeDtypeStruct((N, H), jnp.float32)],
    )(degp, x, W1)

    p1 = layer_k(y1, row, col, w).reshape(NC, N, H)

    y2 = pl.pallas_call(
        _tc2_body,
        out_shape=jax.ShapeDtypeStruct((N, H), jnp.float32),
    )(p1, y1, dis, b1.reshape(1, H), W2)

    p2 = layer_k(y2, row, col, w).reshape(NC, N, H)

    logits2, value = pl.pallas_call(
        _tc3_body,
        out_shape=[jax.ShapeDtypeStruct((N, 1), jnp.float32),
                   jax.ShapeDtypeStruct((G_SEG, 1), jnp.float32)],
    )(p2, y2, dis, b2.reshape(1, H), Wa, ba.reshape(1, 1), Wc,
      bc.reshape(1, 1), batch.reshape(1, N))

    return (logits2.reshape(N), value)
